# baseline (speedup 1.0000x reference)
import functools

import numpy as np
import jax
import jax.numpy as jnp
from jax import lax
from jax.experimental import pallas as pl
from jax.experimental.pallas import tpu as pltpu
from jax.experimental.pallas import tpu_sc as plsc

_D = 64
_SEQ = 2048
_BATCH = 16
_NTOK = 100000

_NC, _NS = 2, 16
_NW = _NC * _NS
_B = _SEQ * _BATCH
_DPW = _D // _NW
_HALF_S = _SEQ // 2
_HALF_I = _HALF_S * _BATCH
_ICH = 8192
_VPC = _ICH // 16
_UNROLL = 8
_BUF_PITCH = _HALF_S + 1


def _pe_table():
    position = np.arange(0, _SEQ, dtype=np.float32)[:, None]
    div_term = np.exp(
        np.arange(0, _D, 2).astype(np.float32) * (-np.log(10000.0) / _D)
    )
    pe = np.zeros((_SEQ, _D), dtype=np.float32)
    pe[:, 0::2] = np.sin(position * div_term)
    pe[:, 1::2] = np.cos(position * div_term)
    return pe


_PE_REP = np.ascontiguousarray(
    np.broadcast_to(_pe_table().T[:, None, :], (_D, _BATCH, _SEQ))
)


def _sc_embed(Wt, xf, pe_rep):
    mesh = plsc.VectorSubcoreMesh(core_axis_name="c", subcore_axis_name="s")

    @functools.partial(
        pl.kernel,
        mesh=mesh,
        out_type=jax.ShapeDtypeStruct((_BATCH, _D, _SEQ), jnp.float32),
        scratch_types=[
            pltpu.VMEM((_NTOK,), jnp.float32),
            pltpu.VMEM((_BATCH, _BUF_PITCH), jnp.float32),
            pltpu.VMEM((_ICH,), jnp.int32),
            pltpu.SemaphoreType.DMA,
        ],
        compiler_params=pltpu.CompilerParams(needs_layout_passes=False),
    )
    def k(wt_hbm, x_hbm, pe_hbm, out_hbm, row_v, buf, idx_v, sem):
        wid = lax.axis_index("s") * _NC + lax.axis_index("c")
        rows = lax.iota(jnp.int32, 16)
        for dd in range(_DPW):
            d = wid * _DPW + dd
            row_cp = pltpu.async_copy(wt_hbm.at[d], row_v, sem)
            for half in range(2):
                pltpu.sync_copy(
                    pe_hbm.at[d, :, pl.ds(half * _HALF_S, _HALF_S)],
                    buf.at[:, pl.ds(0, _HALF_S)],
                )
                if half == 0:
                    row_cp.wait()
                for ic in range(_HALF_I // _ICH):
                    base_i = half * _HALF_I + ic * _ICH
                    pltpu.sync_copy(x_hbm.at[pl.ds(base_i, _ICH)], idx_v)
                    s_base = base_i // _BATCH - half * _HALF_S

                    def body(j, carry, s_base=s_base):
                        for u in range(_UNROLL):
                            v = j * _UNROLL + u
                            iv = idx_v[pl.ds(v * 16, 16)]
                            g = plsc.load_gather(row_v, [iv])
                            col = jnp.full((16,), 0, jnp.int32) + (s_base + v)
                            plsc.addupdate_scatter(buf, [rows, col], g)
                        return carry

                    lax.fori_loop(0, _VPC // _UNROLL, body, 0)
                pltpu.sync_copy(
                    buf.at[:, pl.ds(0, _HALF_S)],
                    out_hbm.at[:, d, pl.ds(half * _HALF_S, _HALF_S)],
                )

    return k(Wt, xf, pe_rep)


def kernel(x, W):
    Wt = W.T
    xf = x.reshape(_B)
    pe_rep = jnp.asarray(_PE_REP)
    out_k = _sc_embed(Wt, xf, pe_rep)
    return out_k.transpose(2, 0, 1)


# speedup vs baseline: 1.5921x; 1.5921x over previous
import functools

import numpy as np
import jax
import jax.numpy as jnp
from jax import lax
from jax.experimental import pallas as pl
from jax.experimental.pallas import tpu as pltpu
from jax.experimental.pallas import tpu_sc as plsc

_D = 64
_SEQ = 2048
_BATCH = 16
_NTOK = 100000

_NC, _NS = 2, 16
_NW = _NC * _NS
_DPW = _D // _NW
_QS = 512
_NQ = _SEQ // _QS
_GPQ = _QS // 16


def _pe_table():
    position = np.arange(0, _SEQ, dtype=np.float32)[:, None]
    div_term = np.exp(
        np.arange(0, _D, 2).astype(np.float32) * (-np.log(10000.0) / _D)
    )
    pe = np.zeros((_SEQ, _D), dtype=np.float32)
    pe[:, 0::2] = np.sin(position * div_term)
    pe[:, 1::2] = np.cos(position * div_term)
    return pe


_PE_T = np.ascontiguousarray(_pe_table().T)


def _sc_embed(Wt, xT, peT):
    mesh = plsc.VectorSubcoreMesh(core_axis_name="c", subcore_axis_name="s")

    @functools.partial(
        pl.kernel,
        mesh=mesh,
        out_type=jax.ShapeDtypeStruct((_BATCH, _D, _SEQ), jnp.float32),
        scratch_types=[
            pltpu.VMEM((_NTOK,), jnp.float32),
            pltpu.VMEM((_BATCH, _QS), jnp.float32),
            pltpu.VMEM((_BATCH, _QS), jnp.int32),
            pltpu.VMEM((_SEQ,), jnp.float32),
            pltpu.SemaphoreType.DMA,
        ],
        compiler_params=pltpu.CompilerParams(needs_layout_passes=False),
    )
    def k(wt_hbm, x_hbm, pe_hbm, out_hbm, row_v, buf, idx_v, pe_v, sem):
        wid = lax.axis_index("s") * _NC + lax.axis_index("c")
        for dd in range(_DPW):
            d = wid * _DPW + dd
            row_cp = pltpu.async_copy(wt_hbm.at[d], row_v, sem)
            pe_cp = pltpu.async_copy(pe_hbm.at[d], pe_v, sem)
            for q in range(_NQ):
                pltpu.sync_copy(x_hbm.at[:, pl.ds(q * _QS, _QS)], idx_v)
                if q == 0:
                    row_cp.wait()
                    pe_cp.wait()

                def body(j, carry, q=q):
                    p = pe_v[pl.ds(q * _QS + j * 16, 16)]
                    for b in range(_BATCH):
                        iv = idx_v[b, pl.ds(j * 16, 16)]
                        g = plsc.load_gather(row_v, [iv])
                        buf[b, pl.ds(j * 16, 16)] = g + p
                    return carry

                lax.fori_loop(0, _GPQ, body, 0)
                pltpu.sync_copy(buf, out_hbm.at[:, d, pl.ds(q * _QS, _QS)])

    return k(Wt, xT, peT)


def kernel(x, W):
    Wt = W.T
    xT = x.T
    peT = jnp.asarray(_PE_T)
    out_k = _sc_embed(Wt, xT, peT)
    return out_k.transpose(2, 0, 1)


# speedup vs baseline: 2.0829x; 1.3083x over previous
import functools

import numpy as np
import jax
import jax.numpy as jnp
from jax import lax
from jax.experimental import pallas as pl
from jax.experimental.pallas import tpu as pltpu
from jax.experimental.pallas import tpu_sc as plsc

_D = 64
_SEQ = 2048
_BATCH = 16
_NTOK = 100000

_NC, _NS = 2, 16
_NW = _NC * _NS
_DPW = _D // _NW
_QS = 512
_NQ = _SEQ // _QS
_GPQ = _QS // 16


def _pe_table():
    position = np.arange(0, _SEQ, dtype=np.float32)[:, None]
    div_term = np.exp(
        np.arange(0, _D, 2).astype(np.float32) * (-np.log(10000.0) / _D)
    )
    pe = np.zeros((_SEQ, _D), dtype=np.float32)
    pe[:, 0::2] = np.sin(position * div_term)
    pe[:, 1::2] = np.cos(position * div_term)
    return pe


_PE_T = np.ascontiguousarray(_pe_table().T)


def _sc_embed(Wt, xT, peT):
    mesh = plsc.VectorSubcoreMesh(core_axis_name="c", subcore_axis_name="s")

    @functools.partial(
        pl.kernel,
        mesh=mesh,
        out_type=jax.ShapeDtypeStruct((_BATCH, _D, _SEQ), jnp.float32),
        scratch_types=[
            pltpu.VMEM((_NTOK,), jnp.float32),
            pltpu.VMEM((_BATCH, _QS), jnp.float32),
            pltpu.VMEM((_BATCH, _QS), jnp.int32),
            pltpu.VMEM((_SEQ,), jnp.float32),
            pltpu.SemaphoreType.DMA,
        ],
        compiler_params=pltpu.CompilerParams(needs_layout_passes=False),
    )
    def k(wt_hbm, x_hbm, pe_hbm, out_hbm, row_v, buf, idx_v, pe_v, sem):
        wid = lax.axis_index("s") * _NC + lax.axis_index("c")
        for dd in range(_DPW):
            d = wid * _DPW + dd
            row_cp = pltpu.async_copy(wt_hbm.at[d], row_v, sem)
            pe_cp = pltpu.async_copy(pe_hbm.at[d], pe_v, sem)
            for q in range(_NQ):
                pltpu.sync_copy(x_hbm.at[:, pl.ds(q * _QS, _QS)], idx_v)
                if q == 0:
                    row_cp.wait()
                    pe_cp.wait()

                @plsc.parallel_loop(0, _QS, step=16, unroll=2)
                def body(s0, q=q):
                    p = pe_v[pl.ds(q * _QS + s0, 16)]
                    for b in range(_BATCH):
                        iv = idx_v[b, pl.ds(s0, 16)]
                        g = plsc.load_gather(row_v, [iv])
                        buf[b, pl.ds(s0, 16)] = g + p
                pltpu.sync_copy(buf, out_hbm.at[:, d, pl.ds(q * _QS, _QS)])

    return k(Wt, xT, peT)


def kernel(x, W):
    Wt = W.T
    xT = x.T
    peT = jnp.asarray(_PE_T)
    out_k = _sc_embed(Wt, xT, peT)
    return out_k.transpose(2, 0, 1)
